# Initial kernel scaffold; baseline (speedup 1.0000x reference)
import jax
import jax.numpy as jnp
from jax.experimental import pallas as pl


def kernel(V0, val, bk_idx):
    raise NotImplementedError("write your pallas kernel here")


# speedup vs baseline: 1.4549x; 1.4549x over previous
import functools

import jax
import jax.numpy as jnp
from jax import lax
from jax.experimental import pallas as pl
from jax.experimental.pallas import tpu as pltpu
from jax.experimental.pallas import tpu_sc as plsc

N_ATOMS = 1_000_000
B_TOTAL = 16384

NC = 2
NS = 16
LANES = 16

HALF = N_ATOMS // NC
DUMP = HALF
SP_WORDS = HALF + 8

CHUNK = B_TOTAL // NS
ROWS = 8
COLS = 128
assert ROWS * COLS == CHUNK

CH_A = 31_256
CH_LAST = HALF - 15 * CH_A
assert CH_LAST == 31_160 and CH_LAST % 8 == 0

_mesh = plsc.VectorSubcoreMesh(
    core_axis_name="c", subcore_axis_name="s", num_cores=NC, num_subcores=NS
)


@functools.partial(
    pl.kernel,
    out_type=jax.ShapeDtypeStruct((N_ATOMS,), jnp.float32),
    mesh=_mesh,
    scratch_types=[
        pltpu.VMEM_SHARED((SP_WORDS,), jnp.float32),
        pltpu.VMEM((ROWS, COLS), jnp.int32),
        pltpu.VMEM((ROWS, COLS), jnp.int32),
        pltpu.VMEM((ROWS, COLS), jnp.float32),
        pltpu.VMEM((LANES,), jnp.int32),
        pltpu.VMEM((LANES,), jnp.float32),
        pltpu.VMEM((CH_A,), jnp.float32),
        pltpu.VMEM((COLS,), jnp.float32),
    ],
)
def _facts_scatter(v0_hbm, idx_hbm, val_hbm, out_hbm,
                   vsh, idx_raw, idx_loc, vals, bidx, bval, vbuf, zbuf):
    c = lax.axis_index("c")
    s = lax.axis_index("s")
    base = c * HALF

    off_a = pl.multiple_of(s * CH_A, 8)

    @pl.when(s < NS - 1)
    def _init_main():
        pltpu.sync_copy(v0_hbm.at[pl.ds(base + off_a, CH_A)], vbuf)
        pltpu.sync_copy(vbuf, vsh.at[pl.ds(off_a, CH_A)])

    @pl.when(s == NS - 1)
    def _init_last():
        pltpu.sync_copy(v0_hbm.at[pl.ds(base + 15 * CH_A, CH_LAST)],
                        vbuf.at[pl.ds(0, CH_LAST)])
        pltpu.sync_copy(vbuf.at[pl.ds(0, CH_LAST)],
                        vsh.at[pl.ds(15 * CH_A, CH_LAST)])

    pltpu.sync_copy(idx_hbm.at[s], idx_raw)
    pltpu.sync_copy(val_hbm.at[s], vals)

    for r in range(ROWS):
        for k in range(COLS // LANES):
            g = idx_raw[r, pl.ds(k * LANES, LANES)]
            local = g - base
            in_range = (local >= 0) & (local < HALF)
            idx_loc[r, pl.ds(k * LANES, LANES)] = jnp.where(in_range, local, DUMP)

    lane = lax.iota(jnp.int32, LANES)
    bidx[...] = jnp.where(lane == 0, 0, DUMP)
    bval[...] = jnp.where(lane == 0, 1.0, 0.0).astype(jnp.float32)
    for k in range(COLS // LANES):
        zbuf[pl.ds(k * LANES, LANES)] = jnp.zeros((LANES,), jnp.float32)

    plsc.subcore_barrier()

    for r in range(ROWS):
        pltpu.sync_copy(vals.at[r], vsh.at[idx_loc.at[r]], add=True)

    @pl.when((c == 0) & (s == 0))
    def _bias():
        pltpu.sync_copy(bval, vsh.at[bidx], add=True)

    pltpu.sync_copy(zbuf, vsh.at[idx_loc.at[ROWS - 1]], add=True)
    pltpu.sync_copy(zbuf.at[pl.ds(0, LANES)], vsh.at[bidx], add=True)

    plsc.subcore_barrier()

    @pl.when(s < NS - 1)
    def _wb_main():
        pltpu.sync_copy(vsh.at[pl.ds(off_a, CH_A)], vbuf)
        pltpu.sync_copy(vbuf, out_hbm.at[pl.ds(base + off_a, CH_A)])

    @pl.when(s == NS - 1)
    def _wb_last():
        pltpu.sync_copy(vsh.at[pl.ds(15 * CH_A, CH_LAST)],
                        vbuf.at[pl.ds(0, CH_LAST)])
        pltpu.sync_copy(vbuf.at[pl.ds(0, CH_LAST)],
                        out_hbm.at[pl.ds(base + 15 * CH_A, CH_LAST)])


@jax.jit
def kernel(V0, val, bk_idx):
    v0_flat = V0.reshape((N_ATOMS,))
    idx = bk_idx.astype(jnp.int32).reshape(NS, ROWS, COLS)
    vals = val.astype(jnp.float32).reshape(NS, ROWS, COLS)
    out = _facts_scatter(v0_flat, idx, vals)
    return out.reshape(1, N_ATOMS)


# speedup vs baseline: 3.7081x; 2.5487x over previous
import functools

import jax
import jax.numpy as jnp
from jax import lax
from jax.experimental import pallas as pl
from jax.experimental.pallas import tpu as pltpu
from jax.experimental.pallas import tpu_sc as plsc

N_ATOMS = 1_000_000
B_TOTAL = 16384

NC = 2
NS = 16
LANES = 16

H0 = 500_096
H1 = N_ATOMS - H0
DUMP = H0
SP_WORDS = H0 + 128

CHUNK = B_TOTAL // NS
ROWS = 8
COLS = 128
assert ROWS * COLS == CHUNK

CH = 31_232
CH0_LAST = H0 - 15 * CH
CH1_LAST = H1 - 15 * CH
assert CH0_LAST % 8 == 0 and CH1_LAST % 8 == 0

_mesh = plsc.VectorSubcoreMesh(
    core_axis_name="c", subcore_axis_name="s", num_cores=NC, num_subcores=NS
)


@functools.partial(
    pl.kernel,
    out_type=jax.ShapeDtypeStruct((1, N_ATOMS), jnp.float32),
    mesh=_mesh,
    scratch_types=[
        pltpu.VMEM_SHARED((SP_WORDS,), jnp.float32),
        pltpu.VMEM((ROWS, COLS), jnp.int32),
        pltpu.VMEM((ROWS, COLS), jnp.int32),
        pltpu.VMEM((ROWS, COLS), jnp.float32),
        pltpu.VMEM((LANES,), jnp.int32),
        pltpu.VMEM((LANES,), jnp.float32),
        pltpu.VMEM((CH0_LAST,), jnp.float32),
        pltpu.VMEM((COLS,), jnp.float32),
    ],
)
def _facts_scatter(v0_hbm, idx_hbm, val_hbm, out_hbm,
                   vsh, idx_raw, idx_loc, vals, bidx, bval, vbuf, zbuf):
    c = lax.axis_index("c")
    s = lax.axis_index("s")
    base = c * H0
    hsize = H0 - c * (H0 - H1)

    off = pl.multiple_of(s * CH, 128)

    @pl.when(s < NS - 1)
    def _init_main():
        pltpu.sync_copy(v0_hbm.at[0, pl.ds(base + off, CH)],
                        vbuf.at[pl.ds(0, CH)])
        pltpu.sync_copy(vbuf.at[pl.ds(0, CH)], vsh.at[pl.ds(off, CH)])

    @pl.when((s == NS - 1) & (c == 0))
    def _init_last0():
        pltpu.sync_copy(v0_hbm.at[0, pl.ds(15 * CH, CH0_LAST)], vbuf)
        pltpu.sync_copy(vbuf, vsh.at[pl.ds(15 * CH, CH0_LAST)])

    @pl.when((s == NS - 1) & (c == 1))
    def _init_last1():
        pltpu.sync_copy(v0_hbm.at[0, pl.ds(H0 + 15 * CH, CH1_LAST)],
                        vbuf.at[pl.ds(0, CH1_LAST)])
        pltpu.sync_copy(vbuf.at[pl.ds(0, CH1_LAST)],
                        vsh.at[pl.ds(15 * CH, CH1_LAST)])

    pltpu.sync_copy(idx_hbm.at[s], idx_raw)
    pltpu.sync_copy(val_hbm.at[s], vals)

    for r in range(ROWS):
        for k in range(COLS // LANES):
            g = idx_raw[r, pl.ds(k * LANES, LANES)]
            local = g - base
            in_range = (local >= 0) & (local < hsize)
            idx_loc[r, pl.ds(k * LANES, LANES)] = jnp.where(in_range, local, DUMP)

    lane = lax.iota(jnp.int32, LANES)
    bidx[...] = jnp.where(lane == 0, 0, DUMP)
    bval[...] = jnp.where(lane == 0, 1.0, 0.0).astype(jnp.float32)
    for k in range(COLS // LANES):
        zbuf[pl.ds(k * LANES, LANES)] = jnp.zeros((LANES,), jnp.float32)

    plsc.subcore_barrier()

    for r in range(ROWS):
        pltpu.sync_copy(vals.at[r], vsh.at[idx_loc.at[r]], add=True)

    @pl.when((c == 0) & (s == 0))
    def _bias():
        pltpu.sync_copy(bval, vsh.at[bidx], add=True)

    pltpu.sync_copy(zbuf, vsh.at[idx_loc.at[ROWS - 1]], add=True)
    pltpu.sync_copy(zbuf.at[pl.ds(0, LANES)], vsh.at[bidx], add=True)

    plsc.subcore_barrier()

    @pl.when(s < NS - 1)
    def _wb_main():
        pltpu.sync_copy(vsh.at[pl.ds(off, CH)], vbuf.at[pl.ds(0, CH)])
        pltpu.sync_copy(vbuf.at[pl.ds(0, CH)],
                        out_hbm.at[0, pl.ds(base + off, CH)])

    @pl.when((s == NS - 1) & (c == 0))
    def _wb_last0():
        pltpu.sync_copy(vsh.at[pl.ds(15 * CH, CH0_LAST)], vbuf)
        pltpu.sync_copy(vbuf, out_hbm.at[0, pl.ds(15 * CH, CH0_LAST)])

    @pl.when((s == NS - 1) & (c == 1))
    def _wb_last1():
        pltpu.sync_copy(vsh.at[pl.ds(15 * CH, CH1_LAST)],
                        vbuf.at[pl.ds(0, CH1_LAST)])
        pltpu.sync_copy(vbuf.at[pl.ds(0, CH1_LAST)],
                        out_hbm.at[0, pl.ds(H0 + 15 * CH, CH1_LAST)])


@jax.jit
def kernel(V0, val, bk_idx):
    idx = bk_idx.astype(jnp.int32).reshape(NS, ROWS, COLS)
    vals = val.astype(jnp.float32).reshape(NS, ROWS, COLS)
    return _facts_scatter(V0, idx, vals)
